# baseline (speedup 1.0000x reference)
import jax
import jax.numpy as jnp
from jax.experimental import pallas as pl
from jax.experimental.pallas import tpu as pltpu


def _selector_body(x_ref, w_ref, b_ref, idx_ref):
    pooled = jnp.mean(x_ref[...], axis=0, keepdims=True)
    logits = jnp.dot(pooled, w_ref[...],
                     preferred_element_type=jnp.float32) + b_ref[...]
    l = logits[0]
    iota = jax.lax.iota(jnp.int32, l.shape[0])
    i0 = jnp.argmax(l).astype(jnp.int32)
    l2 = jnp.where(iota == i0, -jnp.inf, l)
    i1 = jnp.argmax(l2).astype(jnp.int32)
    idx_ref[0] = i0
    idx_ref[1] = i1


def _chain_body(idx_ref, x_ref, g_ref, beta_ref, w1_ref, b1_ref, w2_ref,
                b2_ref, t_ref, o_ref, h_ref):
    i = pl.program_id(0)
    t = pl.program_id(1)
    R = x_ref.shape[0]

    h = jnp.where(i == 0, x_ref[...], h_ref[pl.ds(t * R, R), :])
    mu = jnp.mean(h, axis=1, keepdims=True)
    var = jnp.mean((h - mu) ** 2, axis=1, keepdims=True)
    hn = (h - mu) * jax.lax.rsqrt(var + 1e-5) * g_ref[0] + beta_ref[0]
    a = jnp.tanh(jnp.dot(hn, w1_ref[0],
                         preferred_element_type=jnp.float32) + b1_ref[0])
    p = jnp.dot(a, w2_ref[0], preferred_element_type=jnp.float32) + b2_ref[0]
    p = p * (1.0 + 0.1 * t_ref[0])
    nh = h + 0.3 * p
    h_ref[pl.ds(t * R, R), :] = nh

    @pl.when(i == 1)
    def _():
        o_ref[...] = nh


def kernel(embodied_input, disembodied_input, torsion_field, sel_W, sel_b,
           ln_g, ln_beta, W1, b1, W2, b2, max_active_blocks):
    B, S, D = embodied_input.shape
    NB = sel_b.shape[0]
    BS = B * S

    x0 = embodied_input[0]
    top_idx = pl.pallas_call(
        _selector_body,
        in_specs=[
            pl.BlockSpec(memory_space=pltpu.VMEM),
            pl.BlockSpec(memory_space=pltpu.VMEM),
            pl.BlockSpec(memory_space=pltpu.VMEM),
        ],
        out_specs=pl.BlockSpec(memory_space=pltpu.SMEM),
        out_shape=jax.ShapeDtypeStruct((2,), jnp.int32),
    )(x0, sel_W, sel_b.reshape(1, NB))

    R = 256
    T = BS // R
    S_per_batch = S

    def widx3(i, t, s):
        del t
        return (s[i], 0, 0)

    h = pl.pallas_call(
        _chain_body,
        grid_spec=pltpu.PrefetchScalarGridSpec(
            num_scalar_prefetch=1,
            grid=(2, T),
            in_specs=[
                pl.BlockSpec((R, D), lambda i, t, s: (jnp.where(i == 0, t, 0), 0)),
                pl.BlockSpec((1, 1, D), widx3),
                pl.BlockSpec((1, 1, D), widx3),
                pl.BlockSpec((1, D, D), widx3),
                pl.BlockSpec((1, 1, D), widx3),
                pl.BlockSpec((1, D, D), widx3),
                pl.BlockSpec((1, 1, D), widx3),
                pl.BlockSpec((1, 1, D),
                             lambda i, t, s: (t * R // S_per_batch, 0, 0)),
            ],
            out_specs=pl.BlockSpec((R, D),
                                   lambda i, t, s: (jnp.where(i == 1, t, 0), 0)),
            scratch_shapes=[pltpu.VMEM((BS, D), jnp.float32)],
        ),
        out_shape=jax.ShapeDtypeStruct((BS, D), jnp.float32),
        compiler_params=pltpu.CompilerParams(
            dimension_semantics=("arbitrary", "arbitrary"),
        ),
    )(top_idx, embodied_input.reshape(BS, D), ln_g.reshape(NB, 1, D),
      ln_beta.reshape(NB, 1, D), W1, b1.reshape(NB, 1, D), W2,
      b2.reshape(NB, 1, D), torsion_field.reshape(B, 1, D))
    return h.reshape(B, S, D)


# speedup vs baseline: 1.1870x; 1.1870x over previous
import jax
import jax.numpy as jnp
from jax.experimental import pallas as pl
from jax.experimental.pallas import tpu as pltpu


def _selector_body(x_ref, w_ref, b_ref, idx_ref):
    pooled = jnp.mean(x_ref[...], axis=0, keepdims=True)
    logits = jnp.dot(pooled, w_ref[...],
                     preferred_element_type=jnp.float32) + b_ref[...]
    l = logits[0]
    iota = jax.lax.iota(jnp.int32, l.shape[0])
    i0 = jnp.argmax(l).astype(jnp.int32)
    l2 = jnp.where(iota == i0, -jnp.inf, l)
    i1 = jnp.argmax(l2).astype(jnp.int32)
    idx_ref[0] = i0
    idx_ref[1] = i1


def _block_body(idx_ref, h_ref, g_ref, beta_ref, w1_ref, b1_ref, w2_ref,
                b2_ref, t_ref, o_ref):
    h = h_ref[...]
    mu = jnp.mean(h, axis=1, keepdims=True)
    var = jnp.mean((h - mu) ** 2, axis=1, keepdims=True)
    hn = (h - mu) * jax.lax.rsqrt(var + 1e-5) * g_ref[0] + beta_ref[0]
    a = jnp.tanh(jnp.dot(hn, w1_ref[0],
                         preferred_element_type=jnp.float32) + b1_ref[0])
    p = jnp.dot(a, w2_ref[0], preferred_element_type=jnp.float32) + b2_ref[0]
    p = p * (1.0 + 0.1 * t_ref[0])
    o_ref[...] = h + 0.3 * p


def _block_pass(h, top_idx, step, ln_g, ln_beta, W1, b1, W2, b2, torsion,
                rows_per_tile):
    BS, D = h.shape
    S_per_batch = BS // torsion.shape[0]
    grid = BS // rows_per_tile

    def widx3(t, s):
        del t
        return (s[step], 0, 0)

    NB = ln_g.shape[0]
    B = torsion.shape[0]
    return pl.pallas_call(
        _block_body,
        grid_spec=pltpu.PrefetchScalarGridSpec(
            num_scalar_prefetch=1,
            grid=(grid,),
            in_specs=[
                pl.BlockSpec((rows_per_tile, D), lambda t, s: (t, 0)),
                pl.BlockSpec((1, 1, D), widx3),
                pl.BlockSpec((1, 1, D), widx3),
                pl.BlockSpec((1, D, D), widx3),
                pl.BlockSpec((1, 1, D), widx3),
                pl.BlockSpec((1, D, D), widx3),
                pl.BlockSpec((1, 1, D), widx3),
                pl.BlockSpec((1, 1, D),
                             lambda t, s: (t * rows_per_tile // S_per_batch, 0, 0)),
            ],
            out_specs=pl.BlockSpec((rows_per_tile, D), lambda t, s: (t, 0)),
        ),
        out_shape=jax.ShapeDtypeStruct((BS, D), jnp.float32),
        compiler_params=pltpu.CompilerParams(
            dimension_semantics=("arbitrary",),
        ),
    )(top_idx, h, ln_g.reshape(NB, 1, D), ln_beta.reshape(NB, 1, D), W1,
      b1.reshape(NB, 1, D), W2, b2.reshape(NB, 1, D), torsion.reshape(B, 1, D))


def kernel(embodied_input, disembodied_input, torsion_field, sel_W, sel_b,
           ln_g, ln_beta, W1, b1, W2, b2, max_active_blocks):
    B, S, D = embodied_input.shape
    NB = sel_b.shape[0]

    x0 = embodied_input[0]
    top_idx = pl.pallas_call(
        _selector_body,
        in_specs=[
            pl.BlockSpec(memory_space=pltpu.VMEM),
            pl.BlockSpec(memory_space=pltpu.VMEM),
            pl.BlockSpec(memory_space=pltpu.VMEM),
        ],
        out_specs=pl.BlockSpec(memory_space=pltpu.SMEM),
        out_shape=jax.ShapeDtypeStruct((2,), jnp.int32),
    )(x0, sel_W, sel_b.reshape(1, NB))

    h = embodied_input.reshape(B * S, D)
    for i in range(2):
        h = _block_pass(h, top_idx, i, ln_g, ln_beta, W1, b1, W2, b2,
                        torsion_field, rows_per_tile=1024)
    return h.reshape(B, S, D)
